# Initial kernel scaffold; baseline (speedup 1.0000x reference)
import functools

import jax
import jax.numpy as jnp
from jax import lax
from jax.experimental import pallas as pl
from jax.experimental.pallas import tpu as pltpu
from jax.experimental.pallas import tpu_sc as plsc

N = 320000
C = 10000
F = 128

NC = 2
NS = 16
NW = NC * NS
CHUNK = 128
NCH = N // CHUNK
ROWS_PER_TILE = C // NS


def _sc_body(x_hbm, lab_hbm, zv_hbm, zc_hbm, ones_hbm,
             out_v_hbm, out_c_hbm,
             acc_v, acc_c, rows_v, idx_v, ones_v):
    cid = lax.axis_index("c")
    sid = lax.axis_index("s")
    wid = sid * NC + cid

    r0 = sid * ROWS_PER_TILE
    pltpu.sync_copy(zv_hbm, acc_v.at[pl.ds(r0, ROWS_PER_TILE)])
    pltpu.sync_copy(zc_hbm, acc_c.at[pl.ds(r0, ROWS_PER_TILE)])
    pltpu.sync_copy(ones_hbm, ones_v)
    plsc.subcore_barrier()

    nch_w = jnp.where(wid < NCH % NW, NCH // NW + 1, NCH // NW)

    def body(t, carry):
        j = wid + t * NW
        pltpu.sync_copy(lab_hbm.at[j], idx_v)
        pltpu.sync_copy(x_hbm.at[j], rows_v)
        pltpu.sync_copy(rows_v, acc_v.at[idx_v], add=True)
        pltpu.sync_copy(ones_v, acc_c.at[idx_v], add=True)
        return carry

    lax.fori_loop(0, nch_w, body, 0)
    plsc.subcore_barrier()

    pltpu.sync_copy(acc_v.at[pl.ds(r0, ROWS_PER_TILE)],
                    out_v_hbm.at[cid, pl.ds(r0, ROWS_PER_TILE)])
    pltpu.sync_copy(acc_c.at[pl.ds(r0, ROWS_PER_TILE)],
                    out_c_hbm.at[cid, pl.ds(r0, ROWS_PER_TILE)])


def _sc_segment_sum(x3, lab2, zv, zc, ones):
    mesh = plsc.VectorSubcoreMesh(core_axis_name="c", subcore_axis_name="s",
                                  num_cores=NC, num_subcores=NS)
    return pl.kernel(
        _sc_body,
        out_type=(
            jax.ShapeDtypeStruct((NC, C, F), jnp.float32),
            jax.ShapeDtypeStruct((NC, C, 16), jnp.float32),
        ),
        mesh=mesh,
        scratch_types=[
            pltpu.VMEM_SHARED((C, F), jnp.float32),
            pltpu.VMEM_SHARED((C, 16), jnp.float32),
            pltpu.VMEM((CHUNK, F), jnp.float32),
            pltpu.VMEM((CHUNK,), jnp.int32),
            pltpu.VMEM((CHUNK, 16), jnp.float32),
        ],
    )(x3, lab2, zv, zc, ones)


def _combine_body(v_ref, c_ref, rm_ref, cc_ref, mean_ref, ccout_ref):
    n = c_ref[0][:, 0:1] + c_ref[1][:, 0:1]
    ccf = cc_ref[...].astype(jnp.float32)
    ccnew = ccf + n
    mean_ref[...] = (rm_ref[...] * ccf + v_ref[0] + v_ref[1]) / ccnew
    ccout_ref[...] = cc_ref[...] + n.astype(jnp.int32)


def _combine(vals, cnts, running_mean, class_count):
    nb = 8
    rb = C // nb
    return pl.pallas_call(
        _combine_body,
        grid=(nb,),
        in_specs=[
            pl.BlockSpec((NC, rb, F), lambda i: (0, i, 0)),
            pl.BlockSpec((NC, rb, 16), lambda i: (0, i, 0)),
            pl.BlockSpec((rb, F), lambda i: (i, 0)),
            pl.BlockSpec((rb, 1), lambda i: (i, 0)),
        ],
        out_specs=[
            pl.BlockSpec((rb, F), lambda i: (i, 0)),
            pl.BlockSpec((rb, 1), lambda i: (i, 0)),
        ],
        out_shape=[
            jax.ShapeDtypeStruct((C, F), jnp.float32),
            jax.ShapeDtypeStruct((C, 1), jnp.int32),
        ],
    )(vals, cnts, running_mean, class_count)


def kernel(inputs, labels, running_mean, class_count):
    x3 = inputs.reshape(NCH, CHUNK, F)
    lab2 = labels.reshape(NCH, CHUNK)
    zv = jnp.zeros((ROWS_PER_TILE, F), jnp.float32)
    zc = jnp.zeros((ROWS_PER_TILE, 16), jnp.float32)
    ones = jnp.ones((CHUNK, 16), jnp.float32)
    vals, cnts = _sc_segment_sum(x3, lab2, zv, zc, ones)
    new_mean, cc_new = _combine(vals, cnts, running_mean, class_count)
    return (inputs, new_mean, cc_new)


# speedup vs baseline: 2.5619x; 2.5619x over previous
import jax
import jax.numpy as jnp
from jax import lax
from jax.experimental import pallas as pl
from jax.experimental.pallas import tpu as pltpu
from jax.experimental.pallas import tpu_sc as plsc

N = 320000
C = 10000
F = 128

NC = 2
NS = 16
NW = NC * NS
CHUNK = 128
NCH = N // CHUNK
RPT = 624
TAIL = C - NS * RPT


def _zero_dump_slices(sid):
    r0 = sid * RPT
    out = []
    for k in range(5):
        nr = CHUNK if k < 4 else RPT - 4 * CHUNK
        out.append((r0 + k * CHUNK, nr))
    return out


def _values_body(x_hbm, lab_hbm, zv_hbm, out_v_hbm, acc_v, rows_v, idx_v):
    cid = lax.axis_index("c")
    sid = lax.axis_index("s")
    wid = sid * NC + cid

    pltpu.sync_copy(zv_hbm, rows_v)
    for rr, nr in _zero_dump_slices(sid):
        pltpu.sync_copy(rows_v.at[pl.ds(0, nr)], acc_v.at[pl.ds(rr, nr)])

    @pl.when(sid == NS - 1)
    def _zero_tail():
        pltpu.sync_copy(rows_v.at[pl.ds(0, TAIL)],
                        acc_v.at[pl.ds(NS * RPT, TAIL)])

    plsc.subcore_barrier()

    nch_w = jnp.where(wid < NCH % NW, NCH // NW + 1, NCH // NW)

    def body(t, carry):
        j = wid + t * NW
        pltpu.sync_copy(lab_hbm.at[j, 0], idx_v)
        pltpu.sync_copy(x_hbm.at[j], rows_v)
        pltpu.sync_copy(rows_v, acc_v.at[idx_v], add=True)
        return carry

    lax.fori_loop(0, nch_w, body, 0)
    plsc.subcore_barrier()

    for rr, nr in _zero_dump_slices(sid):
        pltpu.sync_copy(acc_v.at[pl.ds(rr, nr)], rows_v.at[pl.ds(0, nr)])
        pltpu.sync_copy(rows_v.at[pl.ds(0, nr)],
                        out_v_hbm.at[cid, pl.ds(rr, nr)])

    @pl.when(sid == NS - 1)
    def _dump_tail():
        pltpu.sync_copy(acc_v.at[pl.ds(NS * RPT, TAIL)],
                        rows_v.at[pl.ds(0, TAIL)])
        pltpu.sync_copy(rows_v.at[pl.ds(0, TAIL)],
                        out_v_hbm.at[cid, pl.ds(NS * RPT, TAIL)])


def _counts_body(lab_hbm, zc_hbm, ones_hbm, out_c_hbm, acc_c, ones_v, idx_v):
    cid = lax.axis_index("c")
    sid = lax.axis_index("s")
    wid = sid * NC + cid

    pltpu.sync_copy(zc_hbm, ones_v)
    for rr, nr in _zero_dump_slices(sid):
        pltpu.sync_copy(ones_v.at[pl.ds(0, nr)], acc_c.at[pl.ds(rr, nr)])

    @pl.when(sid == NS - 1)
    def _zero_tail():
        pltpu.sync_copy(ones_v.at[pl.ds(0, TAIL)],
                        acc_c.at[pl.ds(NS * RPT, TAIL)])

    pltpu.sync_copy(ones_hbm, ones_v)
    plsc.subcore_barrier()

    nch_w = jnp.where(wid < NCH % NW, NCH // NW + 1, NCH // NW)

    def body(t, carry):
        j = wid + t * NW
        pltpu.sync_copy(lab_hbm.at[j, 0], idx_v)
        pltpu.sync_copy(ones_v, acc_c.at[idx_v], add=True)
        return carry

    lax.fori_loop(0, nch_w, body, 0)
    plsc.subcore_barrier()

    for rr, nr in _zero_dump_slices(sid):
        pltpu.sync_copy(acc_c.at[pl.ds(rr, nr)], ones_v.at[pl.ds(0, nr)])
        pltpu.sync_copy(ones_v.at[pl.ds(0, nr)],
                        out_c_hbm.at[cid, pl.ds(rr, nr)])

    @pl.when(sid == NS - 1)
    def _dump_tail():
        pltpu.sync_copy(acc_c.at[pl.ds(NS * RPT, TAIL)],
                        ones_v.at[pl.ds(0, TAIL)])
        pltpu.sync_copy(ones_v.at[pl.ds(0, TAIL)],
                        out_c_hbm.at[cid, pl.ds(NS * RPT, TAIL)])


def _sc_mesh():
    return plsc.VectorSubcoreMesh(core_axis_name="c", subcore_axis_name="s",
                                  num_cores=NC, num_subcores=NS)


def _sc_values(x3, lab3, zv):
    return pl.kernel(
        _values_body,
        out_type=jax.ShapeDtypeStruct((NC, C, F), jnp.float32),
        mesh=_sc_mesh(),
        scratch_types=[
            pltpu.VMEM_SHARED((C, F), jnp.float32),
            pltpu.VMEM((CHUNK, F), jnp.float32),
            pltpu.VMEM((CHUNK,), jnp.int32),
        ],
    )(x3, lab3, zv)


def _sc_counts(lab3, zc, ones):
    return pl.kernel(
        _counts_body,
        out_type=jax.ShapeDtypeStruct((NC, C, F), jnp.float32),
        mesh=_sc_mesh(),
        scratch_types=[
            pltpu.VMEM_SHARED((C, F), jnp.float32),
            pltpu.VMEM((CHUNK, F), jnp.float32),
            pltpu.VMEM((CHUNK,), jnp.int32),
        ],
    )(lab3, zc, ones)


def _combine_body(v_ref, c_ref, rm_ref, cc_ref, mean_ref, ccout_ref):
    n = c_ref[0][:, 0:1] + c_ref[1][:, 0:1]
    ccf = cc_ref[...].astype(jnp.float32)
    ccnew = ccf + n
    mean_ref[...] = (rm_ref[...] * ccf + v_ref[0] + v_ref[1]) / ccnew
    ccout_ref[...] = cc_ref[...] + n.astype(jnp.int32)


def _combine(vals, cnts, running_mean, class_count):
    nb = 10
    rb = C // nb
    return pl.pallas_call(
        _combine_body,
        grid=(nb,),
        in_specs=[
            pl.BlockSpec((NC, rb, F), lambda i: (0, i, 0)),
            pl.BlockSpec((NC, rb, F), lambda i: (0, i, 0)),
            pl.BlockSpec((rb, F), lambda i: (i, 0)),
            pl.BlockSpec((rb, 1), lambda i: (i, 0)),
        ],
        out_specs=[
            pl.BlockSpec((rb, F), lambda i: (i, 0)),
            pl.BlockSpec((rb, 1), lambda i: (i, 0)),
        ],
        out_shape=[
            jax.ShapeDtypeStruct((C, F), jnp.float32),
            jax.ShapeDtypeStruct((C, 1), jnp.int32),
        ],
    )(vals, cnts, running_mean, class_count)


def kernel(inputs, labels, running_mean, class_count):
    x3 = inputs.reshape(NCH, CHUNK, F)
    lab3 = labels.reshape(NCH, 1, CHUNK)
    zv = jnp.zeros((CHUNK, F), jnp.float32)
    zc = jnp.zeros((CHUNK, F), jnp.float32)
    ones = jnp.ones((CHUNK, F), jnp.float32)
    vals = _sc_values(x3, lab3, zv)
    cnts = _sc_counts(lab3, zc, ones)
    new_mean, cc_new = _combine(vals, cnts, running_mean, class_count)
    return (inputs, new_mean, cc_new)


# speedup vs baseline: 3.1887x; 1.2446x over previous
import jax
import jax.numpy as jnp
from jax import lax
from jax.experimental import pallas as pl
from jax.experimental.pallas import tpu as pltpu
from jax.experimental.pallas import tpu_sc as plsc

N = 320000
C = 10000
F = 128

NC = 2
NS = 16
NW = NC * NS
CHUNK = 128
NCH = N // CHUNK
RPT = 624
TAIL = C - NS * RPT


def _zero_dump_slices(sid):
    r0 = sid * RPT
    out = []
    for k in range(5):
        nr = CHUNK if k < 4 else RPT - 4 * CHUNK
        out.append((r0 + k * CHUNK, nr))
    return out


def _values_body(x_hbm, lab_hbm, zv_hbm, out_v_hbm, acc_v,
                 rows_a, rows_b, idx_a, idx_b, sem_a, sem_b):
    cid = lax.axis_index("c")
    sid = lax.axis_index("s")
    wid = sid * NC + cid

    pltpu.sync_copy(zv_hbm, rows_a)
    for rr, nr in _zero_dump_slices(sid):
        pltpu.sync_copy(rows_a.at[pl.ds(0, nr)], acc_v.at[pl.ds(rr, nr)])

    @pl.when(sid == NS - 1)
    def _zero_tail():
        pltpu.sync_copy(rows_a.at[pl.ds(0, TAIL)],
                        acc_v.at[pl.ds(NS * RPT, TAIL)])

    plsc.subcore_barrier()

    def fill(buf_rows, buf_idx, sem, s):
        j = wid + s * NW

        @pl.when(j < NCH)
        def _real():
            pltpu.async_copy(x_hbm.at[j], buf_rows, sem)
            pltpu.async_copy(lab_hbm.at[j, 0], buf_idx, sem)

        @pl.when(j >= NCH)
        def _pad():
            pltpu.async_copy(zv_hbm, buf_rows, sem)
            pltpu.async_copy(lab_hbm.at[0, 0], buf_idx, sem)

    def drain(buf_rows, buf_idx, sem):
        pltpu.make_async_copy(x_hbm.at[0], buf_rows, sem).wait()
        pltpu.make_async_copy(lab_hbm.at[0, 0], buf_idx, sem).wait()

    fill(rows_a, idx_a, sem_a, 0)

    def body(u, carry):
        fill(rows_b, idx_b, sem_b, 2 * u + 1)
        drain(rows_a, idx_a, sem_a)
        pltpu.sync_copy(rows_a, acc_v.at[idx_a], add=True)
        fill(rows_a, idx_a, sem_a, 2 * u + 2)
        drain(rows_b, idx_b, sem_b)
        pltpu.sync_copy(rows_b, acc_v.at[idx_b], add=True)
        return carry

    lax.fori_loop(0, 40, body, 0)
    drain(rows_a, idx_a, sem_a)
    pltpu.sync_copy(rows_a, acc_v.at[idx_a], add=True)
    plsc.subcore_barrier()

    for rr, nr in _zero_dump_slices(sid):
        pltpu.sync_copy(acc_v.at[pl.ds(rr, nr)], rows_a.at[pl.ds(0, nr)])
        pltpu.sync_copy(rows_a.at[pl.ds(0, nr)],
                        out_v_hbm.at[cid, pl.ds(rr, nr)])

    @pl.when(sid == NS - 1)
    def _dump_tail():
        pltpu.sync_copy(acc_v.at[pl.ds(NS * RPT, TAIL)],
                        rows_a.at[pl.ds(0, TAIL)])
        pltpu.sync_copy(rows_a.at[pl.ds(0, TAIL)],
                        out_v_hbm.at[cid, pl.ds(NS * RPT, TAIL)])


def _counts_body(lab_hbm, zc_hbm, ones_hbm, out_c_hbm, acc_c, ones_v, idx_v):
    cid = lax.axis_index("c")
    sid = lax.axis_index("s")
    wid = sid * NC + cid

    pltpu.sync_copy(zc_hbm, ones_v)
    for rr, nr in _zero_dump_slices(sid):
        pltpu.sync_copy(ones_v.at[pl.ds(0, nr)], acc_c.at[pl.ds(rr, nr)])

    @pl.when(sid == NS - 1)
    def _zero_tail():
        pltpu.sync_copy(ones_v.at[pl.ds(0, TAIL)],
                        acc_c.at[pl.ds(NS * RPT, TAIL)])

    pltpu.sync_copy(ones_hbm, ones_v)
    plsc.subcore_barrier()

    nch_w = jnp.where(wid < NCH % NW, NCH // NW + 1, NCH // NW)

    def body(t, carry):
        j = wid + t * NW
        pltpu.sync_copy(lab_hbm.at[j, 0], idx_v)
        pltpu.sync_copy(ones_v, acc_c.at[idx_v], add=True)
        return carry

    lax.fori_loop(0, nch_w, body, 0)
    plsc.subcore_barrier()

    for rr, nr in _zero_dump_slices(sid):
        pltpu.sync_copy(acc_c.at[pl.ds(rr, nr)], ones_v.at[pl.ds(0, nr)])
        pltpu.sync_copy(ones_v.at[pl.ds(0, nr)],
                        out_c_hbm.at[cid, pl.ds(rr, nr)])

    @pl.when(sid == NS - 1)
    def _dump_tail():
        pltpu.sync_copy(acc_c.at[pl.ds(NS * RPT, TAIL)],
                        ones_v.at[pl.ds(0, TAIL)])
        pltpu.sync_copy(ones_v.at[pl.ds(0, TAIL)],
                        out_c_hbm.at[cid, pl.ds(NS * RPT, TAIL)])


def _sc_mesh():
    return plsc.VectorSubcoreMesh(core_axis_name="c", subcore_axis_name="s",
                                  num_cores=NC, num_subcores=NS)


def _sc_values(x3, lab3, zv):
    return pl.kernel(
        _values_body,
        out_type=jax.ShapeDtypeStruct((NC, C, F), jnp.float32),
        mesh=_sc_mesh(),
        scratch_types=[
            pltpu.VMEM_SHARED((C, F), jnp.float32),
            pltpu.VMEM((CHUNK, F), jnp.float32),
            pltpu.VMEM((CHUNK, F), jnp.float32),
            pltpu.VMEM((CHUNK,), jnp.int32),
            pltpu.VMEM((CHUNK,), jnp.int32),
            pltpu.SemaphoreType.DMA,
            pltpu.SemaphoreType.DMA,
        ],
    )(x3, lab3, zv)


def _sc_counts(lab3, zc, ones):
    return pl.kernel(
        _counts_body,
        out_type=jax.ShapeDtypeStruct((NC, C, F), jnp.float32),
        mesh=_sc_mesh(),
        scratch_types=[
            pltpu.VMEM_SHARED((C, F), jnp.float32),
            pltpu.VMEM((CHUNK, F), jnp.float32),
            pltpu.VMEM((CHUNK,), jnp.int32),
        ],
    )(lab3, zc, ones)


def _combine_body(v_ref, c_ref, rm_ref, cc_ref, mean_ref, ccout_ref):
    n = c_ref[0][:, 0:1] + c_ref[1][:, 0:1]
    ccf = cc_ref[...].astype(jnp.float32)
    ccnew = ccf + n
    mean_ref[...] = (rm_ref[...] * ccf + v_ref[0] + v_ref[1]) / ccnew
    ccout_ref[...] = cc_ref[...] + n.astype(jnp.int32)


def _combine(vals, cnts, running_mean, class_count):
    nb = 10
    rb = C // nb
    return pl.pallas_call(
        _combine_body,
        grid=(nb,),
        in_specs=[
            pl.BlockSpec((NC, rb, F), lambda i: (0, i, 0)),
            pl.BlockSpec((NC, rb, F), lambda i: (0, i, 0)),
            pl.BlockSpec((rb, F), lambda i: (i, 0)),
            pl.BlockSpec((rb, 1), lambda i: (i, 0)),
        ],
        out_specs=[
            pl.BlockSpec((rb, F), lambda i: (i, 0)),
            pl.BlockSpec((rb, 1), lambda i: (i, 0)),
        ],
        out_shape=[
            jax.ShapeDtypeStruct((C, F), jnp.float32),
            jax.ShapeDtypeStruct((C, 1), jnp.int32),
        ],
    )(vals, cnts, running_mean, class_count)


def kernel(inputs, labels, running_mean, class_count):
    x3 = inputs.reshape(NCH, CHUNK, F)
    lab3 = labels.reshape(NCH, 1, CHUNK)
    zv = jnp.zeros((CHUNK, F), jnp.float32)
    zc = jnp.zeros((CHUNK, F), jnp.float32)
    ones = jnp.ones((CHUNK, F), jnp.float32)
    vals = _sc_values(x3, lab3, zv)
    cnts = _sc_counts(lab3, zc, ones)
    new_mean, cc_new = _combine(vals, cnts, running_mean, class_count)
    return (inputs, new_mean, cc_new)


# speedup vs baseline: 3.4044x; 1.0677x over previous
import jax
import jax.numpy as jnp
from jax import lax
from jax.experimental import pallas as pl
from jax.experimental.pallas import tpu as pltpu
from jax.experimental.pallas import tpu_sc as plsc

N = 320000
C = 10000
F = 128

NC = 2
NS = 16
NW = NC * NS
CHUNK = 128
NCH = N // CHUNK
RPT = 624
TAIL = C - NS * RPT


def _zero_dump_slices(sid):
    r0 = sid * RPT
    out = []
    for k in range(5):
        nr = CHUNK if k < 4 else RPT - 4 * CHUNK
        out.append((r0 + k * CHUNK, nr))
    return out


def _values_body(x_hbm, lab_hbm, zv_hbm, out_v_hbm, acc_v,
                 rows_a, rows_b, idx_a, idx_b, sem_a, sem_b):
    cid = lax.axis_index("c")
    sid = lax.axis_index("s")
    wid = sid * NC + cid

    pltpu.sync_copy(zv_hbm, rows_a)
    for rr, nr in _zero_dump_slices(sid):
        pltpu.sync_copy(rows_a.at[pl.ds(0, nr)], acc_v.at[pl.ds(rr, nr)])

    @pl.when(sid == NS - 1)
    def _zero_tail():
        pltpu.sync_copy(rows_a.at[pl.ds(0, TAIL)],
                        acc_v.at[pl.ds(NS * RPT, TAIL)])

    plsc.subcore_barrier()

    def fill(buf_rows, buf_idx, sem, s):
        j = wid + s * NW

        @pl.when(j < NCH)
        def _real():
            pltpu.async_copy(x_hbm.at[j], buf_rows, sem)
            pltpu.async_copy(lab_hbm.at[j, 0], buf_idx, sem)

        @pl.when(j >= NCH)
        def _pad():
            pltpu.async_copy(zv_hbm, buf_rows, sem)
            pltpu.async_copy(lab_hbm.at[0, 0], buf_idx, sem)

    def drain(buf_rows, buf_idx, sem):
        pltpu.make_async_copy(x_hbm.at[0], buf_rows, sem).wait()
        pltpu.make_async_copy(lab_hbm.at[0, 0], buf_idx, sem).wait()

    fill(rows_a, idx_a, sem_a, 0)

    def body(u, carry):
        fill(rows_b, idx_b, sem_b, 2 * u + 1)
        drain(rows_a, idx_a, sem_a)
        pltpu.sync_copy(rows_a, acc_v.at[idx_a], add=True)
        fill(rows_a, idx_a, sem_a, 2 * u + 2)
        drain(rows_b, idx_b, sem_b)
        pltpu.sync_copy(rows_b, acc_v.at[idx_b], add=True)
        return carry

    lax.fori_loop(0, 40, body, 0)
    drain(rows_a, idx_a, sem_a)
    pltpu.sync_copy(rows_a, acc_v.at[idx_a], add=True)
    plsc.subcore_barrier()

    for rr, nr in _zero_dump_slices(sid):
        pltpu.sync_copy(acc_v.at[pl.ds(rr, nr)], rows_a.at[pl.ds(0, nr)])
        pltpu.sync_copy(rows_a.at[pl.ds(0, nr)],
                        out_v_hbm.at[cid, pl.ds(rr, nr)])

    @pl.when(sid == NS - 1)
    def _dump_tail():
        pltpu.sync_copy(acc_v.at[pl.ds(NS * RPT, TAIL)],
                        rows_a.at[pl.ds(0, TAIL)])
        pltpu.sync_copy(rows_a.at[pl.ds(0, TAIL)],
                        out_v_hbm.at[cid, pl.ds(NS * RPT, TAIL)])


def _counts_body(lab_hbm, zc_hbm, ones_hbm, out_c_hbm, acc_c, ones_v,
                 idx_v, idx_b, sem_a, sem_b):
    cid = lax.axis_index("c")
    sid = lax.axis_index("s")
    wid = sid * NC + cid

    pltpu.sync_copy(zc_hbm, ones_v)
    for rr, nr in _zero_dump_slices(sid):
        pltpu.sync_copy(ones_v.at[pl.ds(0, nr)], acc_c.at[pl.ds(rr, nr)])

    @pl.when(sid == NS - 1)
    def _zero_tail():
        pltpu.sync_copy(ones_v.at[pl.ds(0, TAIL)],
                        acc_c.at[pl.ds(NS * RPT, TAIL)])

    pltpu.sync_copy(ones_hbm, ones_v)
    plsc.subcore_barrier()

    def fill(buf_idx, sem, s):
        j = wid + s * NW

        @pl.when(j < NCH)
        def _real():
            pltpu.async_copy(lab_hbm.at[j, 0], buf_idx, sem)

        @pl.when(j >= NCH)
        def _pad():
            pltpu.async_copy(lab_hbm.at[0, 0], buf_idx, sem)

    def drain(buf_idx, sem):
        pltpu.make_async_copy(lab_hbm.at[0, 0], buf_idx, sem).wait()

    fill(idx_v, sem_a, 0)

    def body(u, carry):
        fill(idx_b, sem_b, 2 * u + 1)
        drain(idx_v, sem_a)
        pltpu.sync_copy(ones_v, acc_c.at[idx_v], add=True)
        fill(idx_v, sem_a, 2 * u + 2)
        drain(idx_b, sem_b)
        pltpu.sync_copy(ones_v, acc_c.at[idx_b], add=True)
        return carry

    lax.fori_loop(0, 39, body, 0)
    drain(idx_v, sem_a)

    @pl.when(wid < NCH % NW)
    def _last_slot():
        pltpu.sync_copy(ones_v, acc_c.at[idx_v], add=True)

    plsc.subcore_barrier()

    for rr, nr in _zero_dump_slices(sid):
        pltpu.sync_copy(acc_c.at[pl.ds(rr, nr)], ones_v.at[pl.ds(0, nr)])
        pltpu.sync_copy(ones_v.at[pl.ds(0, nr)],
                        out_c_hbm.at[cid, pl.ds(rr, nr)])

    @pl.when(sid == NS - 1)
    def _dump_tail():
        pltpu.sync_copy(acc_c.at[pl.ds(NS * RPT, TAIL)],
                        ones_v.at[pl.ds(0, TAIL)])
        pltpu.sync_copy(ones_v.at[pl.ds(0, TAIL)],
                        out_c_hbm.at[cid, pl.ds(NS * RPT, TAIL)])


def _sc_mesh():
    return plsc.VectorSubcoreMesh(core_axis_name="c", subcore_axis_name="s",
                                  num_cores=NC, num_subcores=NS)


def _sc_values(x3, lab3, zv):
    return pl.kernel(
        _values_body,
        out_type=jax.ShapeDtypeStruct((NC, C, F), jnp.float32),
        mesh=_sc_mesh(),
        scratch_types=[
            pltpu.VMEM_SHARED((C, F), jnp.float32),
            pltpu.VMEM((CHUNK, F), jnp.float32),
            pltpu.VMEM((CHUNK, F), jnp.float32),
            pltpu.VMEM((CHUNK,), jnp.int32),
            pltpu.VMEM((CHUNK,), jnp.int32),
            pltpu.SemaphoreType.DMA,
            pltpu.SemaphoreType.DMA,
        ],
    )(x3, lab3, zv)


def _sc_counts(lab3, zc, ones):
    return pl.kernel(
        _counts_body,
        out_type=jax.ShapeDtypeStruct((NC, C, F), jnp.float32),
        mesh=_sc_mesh(),
        scratch_types=[
            pltpu.VMEM_SHARED((C, F), jnp.float32),
            pltpu.VMEM((CHUNK, F), jnp.float32),
            pltpu.VMEM((CHUNK,), jnp.int32),
            pltpu.VMEM((CHUNK,), jnp.int32),
            pltpu.SemaphoreType.DMA,
            pltpu.SemaphoreType.DMA,
        ],
    )(lab3, zc, ones)


def _combine_body(v_ref, c_ref, rm_ref, cc_ref, mean_ref, ccout_ref):
    n = c_ref[0][:, 0:1] + c_ref[1][:, 0:1]
    ccf = cc_ref[...].astype(jnp.float32)
    ccnew = ccf + n
    mean_ref[...] = (rm_ref[...] * ccf + v_ref[0] + v_ref[1]) / ccnew
    ccout_ref[...] = cc_ref[...] + n.astype(jnp.int32)


def _combine(vals, cnts, running_mean, class_count):
    nb = 10
    rb = C // nb
    return pl.pallas_call(
        _combine_body,
        grid=(nb,),
        in_specs=[
            pl.BlockSpec((NC, rb, F), lambda i: (0, i, 0)),
            pl.BlockSpec((NC, rb, F), lambda i: (0, i, 0)),
            pl.BlockSpec((rb, F), lambda i: (i, 0)),
            pl.BlockSpec((rb, 1), lambda i: (i, 0)),
        ],
        out_specs=[
            pl.BlockSpec((rb, F), lambda i: (i, 0)),
            pl.BlockSpec((rb, 1), lambda i: (i, 0)),
        ],
        out_shape=[
            jax.ShapeDtypeStruct((C, F), jnp.float32),
            jax.ShapeDtypeStruct((C, 1), jnp.int32),
        ],
    )(vals, cnts, running_mean, class_count)


def kernel(inputs, labels, running_mean, class_count):
    x3 = inputs.reshape(NCH, CHUNK, F)
    lab3 = labels.reshape(NCH, 1, CHUNK)
    zv = jnp.zeros((CHUNK, F), jnp.float32)
    zc = jnp.zeros((CHUNK, F), jnp.float32)
    ones = jnp.ones((CHUNK, F), jnp.float32)
    vals = _sc_values(x3, lab3, zv)
    cnts = _sc_counts(lab3, zc, ones)
    new_mean, cc_new = _combine(vals, cnts, running_mean, class_count)
    return (inputs, new_mean, cc_new)
